# Initial kernel scaffold; baseline (speedup 1.0000x reference)
import jax
import jax.numpy as jnp
from jax.experimental import pallas as pl


def kernel(sequence, token_table):
    raise NotImplementedError("write your pallas kernel here")


# speedup vs baseline: 3.9398x; 3.9398x over previous
import functools

import numpy as np
import jax
import jax.numpy as jnp
from jax import lax
from jax.experimental import pallas as pl
from jax.experimental.pallas import tpu as pltpu
from jax.experimental.pallas import tpu_sc as plsc

VOCAB = 100000
EMBED = 128
MAX_LEN = 512
B, L = 1024, 200

_NUM_CORES = 2
_NUM_SUBCORES = 16
_NW = _NUM_CORES * _NUM_SUBCORES
_ROWS_PER_W = B // _NW
_LANES = 16
_CHUNK0 = 128
_CHUNK1 = L - _CHUNK0


def _sinusoidal_pe(max_len, d_model):
    position = np.arange(max_len, dtype=np.float64)[:, None]
    div_term = np.exp(
        np.arange(0, d_model, 2, dtype=np.float64) * -(np.log(10000.0) / d_model)
    )
    pe = np.zeros((max_len, d_model), dtype=np.float64)
    pe[:, 0::2] = np.sin(position * div_term)
    pe[:, 1::2] = np.cos(position * div_term)
    return pe.astype(np.float32)


_PE = _sinusoidal_pe(MAX_LEN, EMBED)[:L]


def _sc_body(table_hbm, idx_hbm, pe_hbm, out_hbm, idx_v, rows_v, pe_v, sem):
    wid = lax.axis_index("s") * _NUM_CORES + lax.axis_index("c")

    pltpu.sync_copy(pe_hbm, pe_v)

    @pl.loop(0, _ROWS_PER_W)
    def _row(i):
        base = (wid * _ROWS_PER_W + i) * L

        pltpu.sync_copy(idx_hbm.at[pl.ds(base, L)], idx_v)
        g0 = pltpu.async_copy(
            table_hbm.at[idx_v.at[pl.ds(0, _CHUNK0)]],
            rows_v.at[pl.ds(0, _CHUNK0), :],
            sem,
        )
        g1 = pltpu.async_copy(
            table_hbm.at[idx_v.at[pl.ds(_CHUNK0, _CHUNK1)]],
            rows_v.at[pl.ds(_CHUNK0, _CHUNK1), :],
            sem,
        )
        g0.wait()
        g1.wait()

        @pl.loop(0, L)
        def _add(r):
            for d in range(EMBED // _LANES):
                plsc.addupdate(
                    rows_v.at[r, pl.ds(d * _LANES, _LANES)],
                    pe_v[r, pl.ds(d * _LANES, _LANES)],
                )

        pltpu.sync_copy(rows_v, out_hbm.at[pl.ds(base, L), :])


@functools.partial(
    pl.kernel,
    out_type=jax.ShapeDtypeStruct((B * L, EMBED), jnp.float32),
    mesh=plsc.VectorSubcoreMesh(core_axis_name="c", subcore_axis_name="s"),
    scratch_types=[
        pltpu.VMEM((L,), jnp.int32),
        pltpu.VMEM((L, EMBED), jnp.float32),
        pltpu.VMEM((L, EMBED), jnp.float32),
        pltpu.SemaphoreType.DMA,
    ],
)
def _sc_embed(table_hbm, idx_hbm, pe_hbm, out_hbm, idx_v, rows_v, pe_v, sem):
    _sc_body(table_hbm, idx_hbm, pe_hbm, out_hbm, idx_v, rows_v, pe_v, sem)


def kernel(sequence, token_table):
    idx = sequence.reshape(-1).astype(jnp.int32)
    out = _sc_embed(token_table, idx, jnp.asarray(_PE))
    return out.reshape(B, L, EMBED)


# speedup vs baseline: 6.6304x; 1.6829x over previous
import functools

import numpy as np
import jax
import jax.numpy as jnp
from jax import lax
from jax.experimental import pallas as pl
from jax.experimental.pallas import tpu as pltpu
from jax.experimental.pallas import tpu_sc as plsc

VOCAB = 100000
EMBED = 128
MAX_LEN = 512
B, L = 1024, 200

_NUM_CORES = 2
_NUM_SUBCORES = 16
_NW = _NUM_CORES * _NUM_SUBCORES
_LANES = 16
_CH = 128
_RPW = (B * L) // _NW
_CPW = _RPW // _CH
_NBUF = 4
_LOOK = 2


def _sinusoidal_pe(max_len, d_model):
    position = np.arange(max_len, dtype=np.float64)[:, None]
    div_term = np.exp(
        np.arange(0, d_model, 2, dtype=np.float64) * -(np.log(10000.0) / d_model)
    )
    pe = np.zeros((max_len, d_model), dtype=np.float64)
    pe[:, 0::2] = np.sin(position * div_term)
    pe[:, 1::2] = np.cos(position * div_term)
    return pe.astype(np.float32)


_PE = _sinusoidal_pe(MAX_LEN, EMBED)[:L]
_PE2 = np.concatenate([_PE, _PE], axis=0)


def _sc_body(table_hbm, idx_hbm, pe_hbm, out_hbm, idx_v, pe_v, rows, sem_g, sem_s):
    wid = lax.axis_index("s") * _NUM_CORES + lax.axis_index("c")
    base = wid * _RPW

    pltpu.sync_copy(pe_hbm, pe_v)
    pltpu.sync_copy(idx_hbm.at[pl.ds(base, _RPW)], idx_v)

    def gather(c):
        b = c % _NBUF
        return pltpu.make_async_copy(
            table_hbm.at[idx_v.at[pl.ds(c * _CH, _CH)]], rows[b], sem_g[b]
        )

    def store(c):
        b = c % _NBUF
        return pltpu.make_async_copy(
            rows[b], out_hbm.at[pl.ds(base + c * _CH, _CH)], sem_s[b]
        )

    for c in range(_LOOK):
        gather(c).start()

    for c in range(_CPW):
        gather(c).wait()
        b = c % _NBUF
        l0 = (c * _CH) % L

        @pl.loop(0, _CH)
        def _add(r, b=b, l0=l0):
            for d in range(EMBED // _LANES):
                sl = pl.ds(d * _LANES, _LANES)
                plsc.addupdate(rows[b].at[r, sl], pe_v[l0 + r, sl])

        store(c).start()
        p = c + _LOOK
        if p < _CPW:
            if p >= _NBUF:
                store(p - _NBUF).wait()
            gather(p).start()

    for c in range(max(0, _CPW - _NBUF), _CPW):
        store(c).wait()


@functools.partial(
    pl.kernel,
    out_type=jax.ShapeDtypeStruct((B * L, EMBED), jnp.float32),
    mesh=plsc.VectorSubcoreMesh(core_axis_name="c", subcore_axis_name="s"),
    scratch_types=[
        pltpu.VMEM((_RPW,), jnp.int32),
        pltpu.VMEM((2 * L, EMBED), jnp.float32),
        [pltpu.VMEM((_CH, EMBED), jnp.float32) for _ in range(_NBUF)],
        [pltpu.SemaphoreType.DMA for _ in range(_NBUF)],
        [pltpu.SemaphoreType.DMA for _ in range(_NBUF)],
    ],
)
def _sc_embed(table_hbm, idx_hbm, pe_hbm, out_hbm, idx_v, pe_v, rows, sem_g, sem_s):
    _sc_body(table_hbm, idx_hbm, pe_hbm, out_hbm, idx_v, pe_v, rows, sem_g, sem_s)


def kernel(sequence, token_table):
    idx = sequence.reshape(-1).astype(jnp.int32)
    out = _sc_embed(token_table, idx, jnp.asarray(_PE2))
    return out.reshape(B, L, EMBED)
